# baseline (speedup 1.0000x reference)
import functools

import jax
import jax.numpy as jnp
from jax import lax
from jax.experimental import pallas as pl
from jax.experimental.pallas import tpu as pltpu
from jax.experimental.pallas import tpu_sc as plsc

N_NODES = 10000
N_EDGES = 320000
D = 128
N_RELS = 64

NC = 2
NS = 16
NW = NC * NS
EPW = N_EDGES // NW
C = 80
NCH = EPW // C
G = C // 16


def _build():
    mesh = plsc.VectorSubcoreMesh(core_axis_name="c", subcore_axis_name="s")

    @functools.partial(
        pl.kernel,
        mesh=mesh,
        compiler_params=pltpu.CompilerParams(needs_layout_passes=False, use_tc_tiling_on_sc=False),
        out_type=jax.ShapeDtypeStruct((N_EDGES,), jnp.float32),
        scratch_types=[
            pltpu.VMEM((2 * EPW,), jnp.int32),
            pltpu.VMEM((EPW,), jnp.int32),
            pltpu.VMEM_SHARED((N_RELS, D // 2), jnp.int32),
            pltpu.VMEM((2 * C, D // 2), jnp.int32),
            pltpu.VMEM((2 * C, D // 2), jnp.int32),
            pltpu.VMEM((C, D // 2), jnp.int32),
            pltpu.VMEM((C, D // 2), jnp.int32),
            pltpu.VMEM((EPW,), jnp.float32),
            pltpu.VMEM((1296,), jnp.float32),
            pltpu.SemaphoreType.DMA,
            pltpu.SemaphoreType.DMA,
            pltpu.SemaphoreType.DMA,
            pltpu.SemaphoreType.DMA,
        ],
    )
    def scorer(emb, wrel, st, et, out,
               st_v, et_v, w_sh, so0_v, so1_v, w0_v, w1_v,
               out_v, t_v,
               sem_so0, sem_so1, sem_w0, sem_w1):
        wid = lax.axis_index("s") * NC + lax.axis_index("c")
        base = wid * EPW
        pltpu.sync_copy(st.at[pl.ds(2 * base, 2 * EPW)], st_v)
        pltpu.sync_copy(et.at[pl.ds(base, EPW)], et_v)
        @pl.when(lax.axis_index("s") == 0)
        def _init_w():
            pltpu.sync_copy(wrel, w_sh)
        plsc.subcore_barrier()

        sobufs = (so0_v, so1_v)
        wbufs = (w0_v, w1_v)
        sosems = (sem_so0, sem_so1)
        wsems = (sem_w0, sem_w1)

        lane = lax.iota(jnp.int32, 16)

        def start(c, b):
            off = c * C
            pltpu.async_copy(emb.at[st_v.at[pl.ds(2 * off, 2 * C)]],
                             sobufs[b], sosems[b])
            pltpu.async_copy(w_sh.at[et_v.at[pl.ds(off, C)]], wbufs[b], wsems[b])

        def wait(b):
            dummy = emb.at[st_v.at[pl.ds(0, 2 * C)]]
            dummy_w = emb.at[et_v.at[pl.ds(0, C)]]
            pltpu.make_async_copy(dummy, sobufs[b], sosems[b]).wait()
            pltpu.make_async_copy(dummy_w, wbufs[b], wsems[b]).wait()

        lane81 = lane * 81

        def compute(c, b):
            so_v = sobufs[b]
            w_v = wbufs[b]

            @plsc.parallel_loop(0, C, unroll=4)
            def ebody(e):
                pair = []
                for j in range(4):
                    sv32 = plsc.bitcast(so_v[2 * e, pl.ds(j * 16, 16)],
                                        jnp.bfloat16)
                    ov32 = plsc.bitcast(so_v[2 * e + 1, pl.ds(j * 16, 16)],
                                        jnp.bfloat16)
                    wv32 = plsc.bitcast(w_v[e, pl.ds(j * 16, 16)],
                                        jnp.bfloat16)
                    pair.append((sv32 * ov32) * wv32)
                ta, tb = plsc.unpack(pair[0] + pair[1],
                                     format=plsc.PackFormat.INTERLEAVED)
                tc, td = plsc.unpack(pair[2] + pair[3],
                                     format=plsc.PackFormat.INTERLEAVED)
                plsc.store_scatter(t_v, [lane81 + e], (ta + tb) + (tc + td))
            for g in range(G):
                z = jnp.zeros((16,), jnp.float32)
                parts = [z, z, z, z]
                for l in range(16):
                    parts[l % 4] = parts[l % 4] + t_v[pl.ds(l * 81 + g * 16, 16)]
                out_v[pl.ds(c * C + g * 16, 16)] = (
                    (parts[0] + parts[1]) + (parts[2] + parts[3]))

        start(0, 0)
        def pair_body(c2, carry):
            c = c2 * 2
            wait(0)
            start(c + 1, 1)
            compute(c, 0)
            wait(1)
            start(c + 2, 0)
            compute(c + 1, 1)
            return carry

        lax.fori_loop(0, (NCH - 1) // 2, pair_body, jnp.int32(0))
        wait(0)
        compute(NCH - 1, 0)

        pltpu.sync_copy(out_v, out.at[pl.ds(base, EPW)])

    return scorer


_scorer_cache = []


@jax.jit
def kernel(embedding, w_relation, source, target, edge_types):
    if not _scorer_cache:
        _scorer_cache.append(_build())
    emb_packed = jax.lax.bitcast_convert_type(
        embedding.astype(jnp.bfloat16).reshape(N_NODES, D // 2, 2), jnp.int32)
    w_packed = jax.lax.bitcast_convert_type(
        w_relation.astype(jnp.bfloat16).reshape(N_RELS, D // 2, 2),
        jnp.int32)
    st = jnp.stack([source, target], axis=1).reshape(-1)
    return _scorer_cache[0](emb_packed, w_packed,
                            st, edge_types)


# speedup vs baseline: 2.1877x; 2.1877x over previous
import functools

import jax
import jax.numpy as jnp
from jax import lax
from jax.experimental import pallas as pl
from jax.experimental.pallas import tpu as pltpu
from jax.experimental.pallas import tpu_sc as plsc

N_NODES = 10000
N_EDGES = 320000
D = 128
N_RELS = 64

NC = 2
NS = 16
NW = NC * NS
EPW = N_EDGES // NW
C = 80
NCH = EPW // C
G = C // 16


def _build():
    mesh = plsc.VectorSubcoreMesh(core_axis_name="c", subcore_axis_name="s")

    @functools.partial(
        pl.kernel,
        mesh=mesh,
        compiler_params=pltpu.CompilerParams(needs_layout_passes=False, use_tc_tiling_on_sc=False),
        out_type=jax.ShapeDtypeStruct((N_EDGES,), jnp.float32),
        scratch_types=[
            pltpu.VMEM((EPW,), jnp.int32),
            pltpu.VMEM((EPW,), jnp.int32),
            pltpu.VMEM((EPW,), jnp.int32),
            pltpu.VMEM_SHARED((N_RELS, D // 2), jnp.int32),
            pltpu.VMEM((C, D // 2), jnp.int32),
            pltpu.VMEM((C, D // 2), jnp.int32),
            pltpu.VMEM((C, D // 2), jnp.int32),
            pltpu.VMEM((C, D // 2), jnp.int32),
            pltpu.VMEM((C, D // 2), jnp.int32),
            pltpu.VMEM((C, D // 2), jnp.int32),
            pltpu.VMEM((EPW,), jnp.float32),
            pltpu.VMEM((1296,), jnp.float32),
            pltpu.SemaphoreType.DMA,
            pltpu.SemaphoreType.DMA,
            pltpu.SemaphoreType.DMA,
            pltpu.SemaphoreType.DMA,
            pltpu.SemaphoreType.DMA,
            pltpu.SemaphoreType.DMA,
        ],
    )
    def scorer(emb, wrel, src, tgt, et, out,
               src_v, tgt_v, et_v, w_sh, s0_v, s1_v, o0_v, o1_v, w0_v, w1_v,
               out_v, t_v,
               sem_s0, sem_s1, sem_o0, sem_o1, sem_w0, sem_w1):
        wid = lax.axis_index("s") * NC + lax.axis_index("c")
        base = wid * EPW
        pltpu.sync_copy(src.at[pl.ds(base, EPW)], src_v)
        pltpu.sync_copy(tgt.at[pl.ds(base, EPW)], tgt_v)
        pltpu.sync_copy(et.at[pl.ds(base, EPW)], et_v)
        @pl.when(lax.axis_index("s") == 0)
        def _init_w():
            pltpu.sync_copy(wrel, w_sh)
        plsc.subcore_barrier()

        sbufs = (s0_v, s1_v)
        obufs = (o0_v, o1_v)
        wbufs = (w0_v, w1_v)
        ssems = (sem_s0, sem_s1)
        osems = (sem_o0, sem_o1)
        wsems = (sem_w0, sem_w1)

        lane = lax.iota(jnp.int32, 16)

        def start(c, b):
            off = c * C
            pltpu.async_copy(emb.at[src_v.at[pl.ds(off, C)]], sbufs[b], ssems[b])
            pltpu.async_copy(emb.at[tgt_v.at[pl.ds(off, C)]], obufs[b], osems[b])
            pltpu.async_copy(w_sh.at[et_v.at[pl.ds(off, C)]], wbufs[b], wsems[b])

        def wait(b):
            dummy = emb.at[src_v.at[pl.ds(0, C)]]
            pltpu.make_async_copy(dummy, sbufs[b], ssems[b]).wait()
            pltpu.make_async_copy(dummy, obufs[b], osems[b]).wait()
            pltpu.make_async_copy(dummy, wbufs[b], wsems[b]).wait()

        lane81 = lane * 81

        def compute(c, b):
            s_v = sbufs[b]
            o_v = obufs[b]
            w_v = wbufs[b]

            @plsc.parallel_loop(0, C, unroll=8)
            def ebody(e):
                pair = []
                for j in range(4):
                    sv32 = plsc.bitcast(s_v[e, pl.ds(j * 16, 16)],
                                        jnp.bfloat16)
                    ov32 = plsc.bitcast(o_v[e, pl.ds(j * 16, 16)],
                                        jnp.bfloat16)
                    wv32 = plsc.bitcast(w_v[e, pl.ds(j * 16, 16)],
                                        jnp.bfloat16)
                    pair.append((sv32 * ov32) * wv32)
                ta, tb = plsc.unpack(pair[0] + pair[1],
                                     format=plsc.PackFormat.INTERLEAVED)
                tc, td = plsc.unpack(pair[2] + pair[3],
                                     format=plsc.PackFormat.INTERLEAVED)
                plsc.store_scatter(t_v, [lane81 + e], (ta + tb) + (tc + td))
            for g in range(G):
                z = jnp.zeros((16,), jnp.float32)
                parts = [z, z, z, z]
                for l in range(16):
                    parts[l % 4] = parts[l % 4] + t_v[pl.ds(l * 81 + g * 16, 16)]
                out_v[pl.ds(c * C + g * 16, 16)] = (
                    (parts[0] + parts[1]) + (parts[2] + parts[3]))

        start(0, 0)
        def pair_body(c2, carry):
            c = c2 * 2
            wait(0)
            start(c + 1, 1)
            compute(c, 0)
            wait(1)
            start(c + 2, 0)
            compute(c + 1, 1)
            return carry

        lax.fori_loop(0, (NCH - 1) // 2, pair_body, jnp.int32(0))
        wait(0)
        compute(NCH - 1, 0)

        pltpu.sync_copy(out_v, out.at[pl.ds(base, EPW)])

    return scorer


_scorer_cache = []


@jax.jit
def kernel(embedding, w_relation, source, target, edge_types):
    if not _scorer_cache:
        _scorer_cache.append(_build())
    emb_packed = jax.lax.bitcast_convert_type(
        embedding.astype(jnp.bfloat16).reshape(N_NODES, D // 2, 2), jnp.int32)
    w_packed = jax.lax.bitcast_convert_type(
        w_relation.astype(jnp.bfloat16).reshape(N_RELS, D // 2, 2),
        jnp.int32)
    return _scorer_cache[0](emb_packed, w_packed,
                            source, target, edge_types)


# speedup vs baseline: 2.7598x; 1.2615x over previous
import functools

import jax
import jax.numpy as jnp
from jax import lax
from jax.experimental import pallas as pl
from jax.experimental.pallas import tpu as pltpu
from jax.experimental.pallas import tpu_sc as plsc

N_NODES = 10000
N_EDGES = 320000
D = 128
N_RELS = 64

NC = 2
NS = 16
NW = NC * NS
EPW = N_EDGES // NW
C = 80
NCH = EPW // C
G = C // 16


def _build():
    mesh = plsc.VectorSubcoreMesh(core_axis_name="c", subcore_axis_name="s")

    @functools.partial(
        pl.kernel,
        mesh=mesh,
        compiler_params=pltpu.CompilerParams(needs_layout_passes=False, use_tc_tiling_on_sc=False),
        out_type=jax.ShapeDtypeStruct((N_EDGES,), jnp.float32),
        scratch_types=[
            pltpu.VMEM((EPW,), jnp.int32),
            pltpu.VMEM((EPW,), jnp.int32),
            pltpu.VMEM((EPW,), jnp.int32),
            pltpu.VMEM_SHARED((N_RELS, D // 2), jnp.int32),
            pltpu.VMEM((C, D // 2), jnp.int32),
            pltpu.VMEM((C, D // 2), jnp.int32),
            pltpu.VMEM((C, D // 2), jnp.int32),
            pltpu.VMEM((C, D // 2), jnp.int32),
            pltpu.VMEM((C, D // 2), jnp.int32),
            pltpu.VMEM((C, D // 2), jnp.int32),
            pltpu.VMEM((C, D // 2), jnp.int32),
            pltpu.VMEM((C, D // 2), jnp.int32),
            pltpu.VMEM((C, D // 2), jnp.int32),
            pltpu.VMEM((EPW,), jnp.float32),
            pltpu.VMEM((1296,), jnp.float32),
            pltpu.SemaphoreType.DMA,
            pltpu.SemaphoreType.DMA,
            pltpu.SemaphoreType.DMA,
            pltpu.SemaphoreType.DMA,
            pltpu.SemaphoreType.DMA,
            pltpu.SemaphoreType.DMA,
            pltpu.SemaphoreType.DMA,
            pltpu.SemaphoreType.DMA,
            pltpu.SemaphoreType.DMA,
        ],
    )
    def scorer(emb, wrel, src, tgt, et, out,
               src_v, tgt_v, et_v, w_sh, s0_v, s1_v, s2_v, o0_v, o1_v, o2_v,
               w0_v, w1_v, w2_v, out_v, t_v,
               sem_s0, sem_s1, sem_s2, sem_o0, sem_o1, sem_o2,
               sem_w0, sem_w1, sem_w2):
        wid = lax.axis_index("s") * NC + lax.axis_index("c")
        base = wid * EPW
        pltpu.sync_copy(src.at[pl.ds(base, EPW)], src_v)
        pltpu.sync_copy(tgt.at[pl.ds(base, EPW)], tgt_v)
        pltpu.sync_copy(et.at[pl.ds(base, EPW)], et_v)
        @pl.when(lax.axis_index("s") == 0)
        def _init_w():
            pltpu.sync_copy(wrel, w_sh)
        plsc.subcore_barrier()

        sbufs = (s0_v, s1_v, s2_v)
        obufs = (o0_v, o1_v, o2_v)
        wbufs = (w0_v, w1_v, w2_v)
        ssems = (sem_s0, sem_s1, sem_s2)
        osems = (sem_o0, sem_o1, sem_o2)
        wsems = (sem_w0, sem_w1, sem_w2)

        lane = lax.iota(jnp.int32, 16)

        def start(c, b):
            off = c * C
            pltpu.async_copy(emb.at[src_v.at[pl.ds(off, C)]], sbufs[b], ssems[b])
            pltpu.async_copy(emb.at[tgt_v.at[pl.ds(off, C)]], obufs[b], osems[b])
            pltpu.async_copy(w_sh.at[et_v.at[pl.ds(off, C)]], wbufs[b], wsems[b])

        def wait(b):
            dummy = emb.at[src_v.at[pl.ds(0, C)]]
            pltpu.make_async_copy(dummy, sbufs[b], ssems[b]).wait()
            pltpu.make_async_copy(dummy, obufs[b], osems[b]).wait()
            pltpu.make_async_copy(dummy, wbufs[b], wsems[b]).wait()

        lane81 = lane * 81

        def compute(c, b):
            s_v = sbufs[b]
            o_v = obufs[b]
            w_v = wbufs[b]

            @plsc.parallel_loop(0, C, unroll=4)
            def ebody(e):
                pair = []
                for j in range(4):
                    sv32 = plsc.bitcast(s_v[e, pl.ds(j * 16, 16)],
                                        jnp.bfloat16)
                    ov32 = plsc.bitcast(o_v[e, pl.ds(j * 16, 16)],
                                        jnp.bfloat16)
                    wv32 = plsc.bitcast(w_v[e, pl.ds(j * 16, 16)],
                                        jnp.bfloat16)
                    pair.append((sv32 * ov32) * wv32)
                ta, tb = plsc.unpack(pair[0] + pair[1],
                                     format=plsc.PackFormat.INTERLEAVED)
                tc, td = plsc.unpack(pair[2] + pair[3],
                                     format=plsc.PackFormat.INTERLEAVED)
                plsc.store_scatter(t_v, [lane81 + e], (ta + tb) + (tc + td))
            for g in range(G):
                z = jnp.zeros((16,), jnp.float32)
                parts = [z, z, z, z]
                for l in range(16):
                    parts[l % 4] = parts[l % 4] + t_v[pl.ds(l * 81 + g * 16, 16)]
                out_v[pl.ds(c * C + g * 16, 16)] = (
                    (parts[0] + parts[1]) + (parts[2] + parts[3]))

        start(0, 0)
        start(1, 1)
        def trip_body(c3, carry):
            c = c3 * 3
            for b in range(3):
                wait(b)
                start(c + b + 2, (b + 2) % 3)
                compute(c + b, b)
            return carry

        lax.fori_loop(0, NCH // 3, trip_body, jnp.int32(0))
        wait((NCH - 2) % 3)
        compute(NCH - 2, (NCH - 2) % 3)
        wait((NCH - 1) % 3)
        compute(NCH - 1, (NCH - 1) % 3)

        pltpu.sync_copy(out_v, out.at[pl.ds(base, EPW)])

    return scorer


_scorer_cache = []


@jax.jit
def kernel(embedding, w_relation, source, target, edge_types):
    if not _scorer_cache:
        _scorer_cache.append(_build())
    emb_packed = jax.lax.bitcast_convert_type(
        embedding.astype(jnp.bfloat16).reshape(N_NODES, D // 2, 2), jnp.int32)
    w_packed = jax.lax.bitcast_convert_type(
        w_relation.astype(jnp.bfloat16).reshape(N_RELS, D // 2, 2),
        jnp.int32)
    return _scorer_cache[0](emb_packed, w_packed,
                            source, target, edge_types)


# speedup vs baseline: 2.7599x; 1.0000x over previous
import functools

import jax
import jax.numpy as jnp
from jax import lax
from jax.experimental import pallas as pl
from jax.experimental.pallas import tpu as pltpu
from jax.experimental.pallas import tpu_sc as plsc

N_NODES = 10000
N_EDGES = 320000
D = 128
N_RELS = 64

NC = 2
NS = 16
NW = NC * NS
EPW = N_EDGES // NW
C = 80
NCH = EPW // C
G = C // 16


def _build():
    mesh = plsc.VectorSubcoreMesh(core_axis_name="c", subcore_axis_name="s")

    @functools.partial(
        pl.kernel,
        mesh=mesh,
        compiler_params=pltpu.CompilerParams(needs_layout_passes=False, use_tc_tiling_on_sc=False),
        out_type=jax.ShapeDtypeStruct((N_EDGES,), jnp.float32),
        scratch_types=[
            pltpu.VMEM((EPW,), jnp.int32),
            pltpu.VMEM((EPW,), jnp.int32),
            pltpu.VMEM((EPW,), jnp.int32),
            pltpu.VMEM_SHARED((N_RELS, D // 2), jnp.int32),
            pltpu.VMEM((C, D // 2), jnp.int32),
            pltpu.VMEM((C, D // 2), jnp.int32),
            pltpu.VMEM((C, D // 2), jnp.int32),
            pltpu.VMEM((C, D // 2), jnp.int32),
            pltpu.VMEM((C, D // 2), jnp.int32),
            pltpu.VMEM((C, D // 2), jnp.int32),
            pltpu.VMEM((C, D // 2), jnp.int32),
            pltpu.VMEM((C, D // 2), jnp.int32),
            pltpu.VMEM((C, D // 2), jnp.int32),
            pltpu.VMEM((C, D // 2), jnp.int32),
            pltpu.VMEM((C, D // 2), jnp.int32),
            pltpu.VMEM((C, D // 2), jnp.int32),
            pltpu.VMEM((EPW,), jnp.float32),
            pltpu.VMEM((1296,), jnp.float32),
            pltpu.SemaphoreType.DMA,
            pltpu.SemaphoreType.DMA,
            pltpu.SemaphoreType.DMA,
            pltpu.SemaphoreType.DMA,
            pltpu.SemaphoreType.DMA,
            pltpu.SemaphoreType.DMA,
            pltpu.SemaphoreType.DMA,
            pltpu.SemaphoreType.DMA,
            pltpu.SemaphoreType.DMA,
            pltpu.SemaphoreType.DMA,
            pltpu.SemaphoreType.DMA,
            pltpu.SemaphoreType.DMA,
        ],
    )
    def scorer(emb, wrel, src, tgt, et, out,
               src_v, tgt_v, et_v, w_sh, s0_v, s1_v, s2_v, s3_v,
               o0_v, o1_v, o2_v, o3_v, w0_v, w1_v, w2_v, w3_v, out_v, t_v,
               sem_s0, sem_s1, sem_s2, sem_s3, sem_o0, sem_o1, sem_o2, sem_o3,
               sem_w0, sem_w1, sem_w2, sem_w3):
        wid = lax.axis_index("s") * NC + lax.axis_index("c")
        base = wid * EPW
        pltpu.sync_copy(src.at[pl.ds(base, EPW)], src_v)
        pltpu.sync_copy(tgt.at[pl.ds(base, EPW)], tgt_v)
        pltpu.sync_copy(et.at[pl.ds(base, EPW)], et_v)
        @pl.when(lax.axis_index("s") == 0)
        def _init_w():
            pltpu.sync_copy(wrel, w_sh)
        plsc.subcore_barrier()

        sbufs = (s0_v, s1_v, s2_v, s3_v)
        obufs = (o0_v, o1_v, o2_v, o3_v)
        wbufs = (w0_v, w1_v, w2_v, w3_v)
        ssems = (sem_s0, sem_s1, sem_s2, sem_s3)
        osems = (sem_o0, sem_o1, sem_o2, sem_o3)
        wsems = (sem_w0, sem_w1, sem_w2, sem_w3)

        lane = lax.iota(jnp.int32, 16)

        def start(c, b):
            off = c * C
            pltpu.async_copy(emb.at[src_v.at[pl.ds(off, C)]], sbufs[b], ssems[b])
            pltpu.async_copy(emb.at[tgt_v.at[pl.ds(off, C)]], obufs[b], osems[b])
            pltpu.async_copy(w_sh.at[et_v.at[pl.ds(off, C)]], wbufs[b], wsems[b])

        def wait(b):
            dummy = emb.at[src_v.at[pl.ds(0, C)]]
            pltpu.make_async_copy(dummy, sbufs[b], ssems[b]).wait()
            pltpu.make_async_copy(dummy, obufs[b], osems[b]).wait()
            pltpu.make_async_copy(dummy, wbufs[b], wsems[b]).wait()

        lane81 = lane * 81

        def compute(c, b):
            s_v = sbufs[b]
            o_v = obufs[b]
            w_v = wbufs[b]

            @plsc.parallel_loop(0, C, unroll=4)
            def ebody(e):
                pair = []
                for j in range(4):
                    sv32 = plsc.bitcast(s_v[e, pl.ds(j * 16, 16)],
                                        jnp.bfloat16)
                    ov32 = plsc.bitcast(o_v[e, pl.ds(j * 16, 16)],
                                        jnp.bfloat16)
                    wv32 = plsc.bitcast(w_v[e, pl.ds(j * 16, 16)],
                                        jnp.bfloat16)
                    pair.append((sv32 * ov32) * wv32)
                ta, tb = plsc.unpack(pair[0] + pair[1],
                                     format=plsc.PackFormat.INTERLEAVED)
                tc, td = plsc.unpack(pair[2] + pair[3],
                                     format=plsc.PackFormat.INTERLEAVED)
                plsc.store_scatter(t_v, [lane81 + e], (ta + tb) + (tc + td))
            for g in range(G):
                z = jnp.zeros((16,), jnp.float32)
                parts = [z, z, z, z]
                for l in range(16):
                    parts[l % 4] = parts[l % 4] + t_v[pl.ds(l * 81 + g * 16, 16)]
                out_v[pl.ds(c * C + g * 16, 16)] = (
                    (parts[0] + parts[1]) + (parts[2] + parts[3]))

        start(0, 0)
        start(1, 1)
        start(2, 2)
        def quad_body(c4, carry):
            c = c4 * 4
            for b in range(4):
                wait(b)
                nxt = c + b + 3
                @pl.when(nxt < NCH)
                def _start_next():
                    start(nxt, (b + 3) % 4)
                compute(c + b, b)
            return carry

        lax.fori_loop(0, NCH // 4, quad_body, jnp.int32(0))
        wait((NCH - 1) % 4)
        compute(NCH - 1, (NCH - 1) % 4)

        pltpu.sync_copy(out_v, out.at[pl.ds(base, EPW)])

    return scorer


_scorer_cache = []


@jax.jit
def kernel(embedding, w_relation, source, target, edge_types):
    if not _scorer_cache:
        _scorer_cache.append(_build())
    emb_packed = jax.lax.bitcast_convert_type(
        embedding.astype(jnp.bfloat16).reshape(N_NODES, D // 2, 2), jnp.int32)
    w_packed = jax.lax.bitcast_convert_type(
        w_relation.astype(jnp.bfloat16).reshape(N_RELS, D // 2, 2),
        jnp.int32)
    return _scorer_cache[0](emb_packed, w_packed,
                            source, target, edge_types)


# speedup vs baseline: 2.7642x; 1.0015x over previous
import functools

import jax
import jax.numpy as jnp
from jax import lax
from jax.experimental import pallas as pl
from jax.experimental.pallas import tpu as pltpu
from jax.experimental.pallas import tpu_sc as plsc

N_NODES = 10000
N_EDGES = 320000
D = 128
N_RELS = 64

NC = 2
NS = 16
NW = NC * NS
EPW = N_EDGES // NW
C = 80
NCH = EPW // C
G = C // 16


def _build():
    mesh = plsc.VectorSubcoreMesh(core_axis_name="c", subcore_axis_name="s")

    @functools.partial(
        pl.kernel,
        mesh=mesh,
        compiler_params=pltpu.CompilerParams(needs_layout_passes=False, use_tc_tiling_on_sc=False),
        out_type=jax.ShapeDtypeStruct((N_EDGES,), jnp.float32),
        scratch_types=[
            pltpu.VMEM((EPW,), jnp.int32),
            pltpu.VMEM((EPW,), jnp.int32),
            pltpu.VMEM((EPW,), jnp.int32),
            pltpu.VMEM_SHARED((N_RELS, D // 2), jnp.int32),
            pltpu.VMEM((C, D // 2), jnp.int32),
            pltpu.VMEM((C, D // 2), jnp.int32),
            pltpu.VMEM((C, D // 2), jnp.int32),
            pltpu.VMEM((C, D // 2), jnp.int32),
            pltpu.VMEM((C, D // 2), jnp.int32),
            pltpu.VMEM((C, D // 2), jnp.int32),
            pltpu.VMEM((C, D // 2), jnp.int32),
            pltpu.VMEM((C, D // 2), jnp.int32),
            pltpu.VMEM((C, D // 2), jnp.int32),
            pltpu.VMEM((EPW,), jnp.float32),
            pltpu.VMEM((1296,), jnp.float32),
            pltpu.SemaphoreType.DMA,
            pltpu.SemaphoreType.DMA,
            pltpu.SemaphoreType.DMA,
            pltpu.SemaphoreType.DMA,
            pltpu.SemaphoreType.DMA,
            pltpu.SemaphoreType.DMA,
            pltpu.SemaphoreType.DMA,
            pltpu.SemaphoreType.DMA,
            pltpu.SemaphoreType.DMA,
        ],
    )
    def scorer(emb, wrel, src, tgt, et, out,
               src_v, tgt_v, et_v, w_sh, s0_v, s1_v, s2_v, o0_v, o1_v, o2_v,
               w0_v, w1_v, w2_v, out_v, t_v,
               sem_s0, sem_s1, sem_s2, sem_o0, sem_o1, sem_o2,
               sem_w0, sem_w1, sem_w2):
        wid = lax.axis_index("s") * NC + lax.axis_index("c")
        base = wid * EPW
        pltpu.sync_copy(src.at[pl.ds(base, EPW)], src_v)
        pltpu.sync_copy(tgt.at[pl.ds(base, EPW)], tgt_v)
        pltpu.sync_copy(et.at[pl.ds(base, EPW)], et_v)
        @pl.when(lax.axis_index("s") == 0)
        def _init_w():
            pltpu.sync_copy(wrel, w_sh)
        plsc.subcore_barrier()

        sbufs = (s0_v, s1_v, s2_v)
        obufs = (o0_v, o1_v, o2_v)
        wbufs = (w0_v, w1_v, w2_v)
        ssems = (sem_s0, sem_s1, sem_s2)
        osems = (sem_o0, sem_o1, sem_o2)
        wsems = (sem_w0, sem_w1, sem_w2)

        lane = lax.iota(jnp.int32, 16)

        def start(c, b):
            off = c * C
            pltpu.async_copy(emb.at[src_v.at[pl.ds(off, C)]], sbufs[b], ssems[b])
            pltpu.async_copy(emb.at[tgt_v.at[pl.ds(off, C)]], obufs[b], osems[b])
            pltpu.async_copy(w_sh.at[et_v.at[pl.ds(off, C)]], wbufs[b], wsems[b])

        def wait(b):
            dummy = emb.at[src_v.at[pl.ds(0, C)]]
            pltpu.make_async_copy(dummy, sbufs[b], ssems[b]).wait()
            pltpu.make_async_copy(dummy, obufs[b], osems[b]).wait()
            pltpu.make_async_copy(dummy, wbufs[b], wsems[b]).wait()

        lane81 = lane * 81

        def compute(c, b):
            s_v = sbufs[b]
            o_v = obufs[b]
            w_v = wbufs[b]

            @plsc.parallel_loop(0, C, unroll=4)
            def ebody(e):
                pair = []
                for j in range(4):
                    sv32 = plsc.bitcast(s_v[e, pl.ds(j * 16, 16)],
                                        jnp.bfloat16)
                    ov32 = plsc.bitcast(o_v[e, pl.ds(j * 16, 16)],
                                        jnp.bfloat16)
                    wv32 = plsc.bitcast(w_v[e, pl.ds(j * 16, 16)],
                                        jnp.bfloat16)
                    pair.append((sv32 * ov32) * wv32)
                ta, tb = plsc.unpack(pair[0] + pair[1],
                                     format=plsc.PackFormat.INTERLEAVED)
                tc, td = plsc.unpack(pair[2] + pair[3],
                                     format=plsc.PackFormat.INTERLEAVED)
                plsc.store_scatter(t_v, [lane81 + e], (ta + tb) + (tc + td))
            for g in range(G):
                z = jnp.zeros((16,), jnp.float32)
                parts = [z, z, z, z]
                for l in range(16):
                    parts[l % 4] = parts[l % 4] + t_v[pl.ds(l * 81 + g * 16, 16)]
                out_v[pl.ds(c * C + g * 16, 16)] = (
                    (parts[0] + parts[1]) + (parts[2] + parts[3]))

        start(0, 0)
        start(1, 1)
        def trip_body(c3, carry):
            c = c3 * 3
            for b in range(3):
                wait(b)
                start(c + b + 2, (b + 2) % 3)
                compute(c + b, b)
            return carry

        lax.fori_loop(0, NCH // 3, trip_body, jnp.int32(0))
        wait((NCH - 2) % 3)
        compute(NCH - 2, (NCH - 2) % 3)
        wait((NCH - 1) % 3)
        compute(NCH - 1, (NCH - 1) % 3)

        pltpu.sync_copy(out_v, out.at[pl.ds(base, EPW)])

    return scorer


_scorer_cache = []


@jax.jit
def kernel(embedding, w_relation, source, target, edge_types):
    if not _scorer_cache:
        _scorer_cache.append(_build())
    emb_packed = jax.lax.bitcast_convert_type(
        embedding.astype(jnp.bfloat16).reshape(N_NODES, D // 2, 2), jnp.int32)
    w_packed = jax.lax.bitcast_convert_type(
        w_relation.astype(jnp.bfloat16).reshape(N_RELS, D // 2, 2),
        jnp.int32)
    return _scorer_cache[0](emb_packed, w_packed,
                            source, target, edge_types)
